# baseline (speedup 1.0000x reference)
import functools
import math

import jax
import jax.numpy as jnp
from jax import lax
from jax.experimental import pallas as pl
from jax.experimental.pallas import tpu as pltpu
from jax.experimental.pallas import tpu_sc as plsc

_SIZE = 32000
_N_TOK = 4096
_SMOOTHING = 0.1
_CONF = 1.0 - _SMOOTHING
_EPS = _SMOOTHING / (_SIZE - 2)
_C_ROW = (_SIZE - 2) * _EPS * math.log(_EPS) + _CONF * math.log(_CONF)

_BR = 512
_BC = 6400

_NC = 2
_NS = 16
_NW = _NC * _NS
_B_PER_W = _N_TOK // _NW


_sc_mesh = plsc.VectorSubcoreMesh(core_axis_name="c", subcore_axis_name="s")


@functools.partial(
    pl.kernel,
    mesh=_sc_mesh,
    out_type=jax.ShapeDtypeStruct((_NW * 16,), jnp.float32),
    scratch_types=[
        pltpu.VMEM((_B_PER_W,), jnp.int32),
        pltpu.VMEM((_B_PER_W,), jnp.float32),
        pltpu.VMEM((16,), jnp.float32),
        pltpu.SemaphoreType.DMA,
    ],
)
def _sc_gather(xflat_hbm, tgt_hbm, out_hbm, idx_v, g_v, acc_v, sem):
    wid = lax.axis_index("s") * _NC + lax.axis_index("c")
    base = wid * _B_PER_W
    pltpu.sync_copy(tgt_hbm.at[pl.ds(base, _B_PER_W)], idx_v)
    for j in range(_B_PER_W // 16):
        t = idx_v[pl.ds(j * 16, 16)]
        row = base + j * 16 + lax.iota(jnp.int32, 16)
        idx_v[pl.ds(j * 16, 16)] = row * _SIZE + t
    pltpu.async_copy(xflat_hbm.at[idx_v], g_v, sem).wait()
    acc = jnp.zeros((16,), jnp.float32)
    for j in range(_B_PER_W // 16):
        idx = idx_v[pl.ds(j * 16, 16)]
        row = base + j * 16 + lax.iota(jnp.int32, 16)
        pad = idx == row * _SIZE
        acc = acc + jnp.where(pad, jnp.zeros((16,), jnp.float32),
                              g_v[pl.ds(j * 16, 16)])
    acc_v[...] = acc
    pltpu.sync_copy(acc_v, out_hbm.at[pl.ds(wid * 16, 16)])


def _tc_body(x_ref, t_ref, out_ref):
    i = pl.program_id(0)
    j = pl.program_id(1)

    @pl.when((i == 0) & (j == 0))
    def _init():
        out_ref[0, 0] = 0.0

    t = t_ref[...]
    mask = t != 0
    x = x_ref[...]

    s = jnp.sum(jnp.where(mask, x, 0.0))
    out_ref[0, 0] += -_EPS * s

    @pl.when(j == 0)
    def _col0():
        x0 = x[:, 0:1]
        out_ref[0, 0] += jnp.sum(jnp.where(mask, _C_ROW + _EPS * x0, 0.0))


def _combine_body(p_ref, sc_ref, out_ref):
    out_ref[0, 0] = p_ref[0, 0] + (_EPS - _CONF) * jnp.sum(sc_ref[...])


def kernel(x, target):
    sc_part = _sc_gather(x.reshape(-1), target)

    t2 = target.reshape(_N_TOK, 1)
    partial = pl.pallas_call(
        _tc_body,
        grid=(_N_TOK // _BR, _SIZE // _BC),
        in_specs=[
            pl.BlockSpec((_BR, _BC), lambda i, j: (i, j)),
            pl.BlockSpec((_BR, 1), lambda i, j: (i, 0)),
        ],
        out_specs=pl.BlockSpec((1, 1), lambda i, j: (0, 0),
                               memory_space=pltpu.SMEM),
        out_shape=jax.ShapeDtypeStruct((1, 1), jnp.float32),
    )(x, t2)

    loss = pl.pallas_call(
        _combine_body,
        in_specs=[
            pl.BlockSpec(memory_space=pltpu.SMEM),
            pl.BlockSpec((4, 128), lambda: (0, 0)),
        ],
        out_specs=pl.BlockSpec(memory_space=pltpu.SMEM),
        out_shape=jax.ShapeDtypeStruct((1, 1), jnp.float32),
    )(partial, sc_part.reshape(4, 128))
    return loss[0, 0]


# speedup vs baseline: 2.9288x; 2.9288x over previous
import math

import jax
import jax.numpy as jnp
from jax.experimental import pallas as pl
from jax.experimental.pallas import tpu as pltpu

_SIZE = 32000
_N_TOK = 4096
_SMOOTHING = 0.1
_CONF = 1.0 - _SMOOTHING
_EPS = _SMOOTHING / (_SIZE - 2)
_C_ROW = (_SIZE - 2) * _EPS * math.log(_EPS) + _CONF * math.log(_CONF)

_BR = 512
_BC = 6400


def _loss_body(x_ref, t_ref, wf_ref, wc_ref, mf_ref, out_ref):
    i = pl.program_id(0)
    j = pl.program_id(1)

    @pl.when((i == 0) & (j == 0))
    def _init():
        out_ref[0, 0] = 0.0

    x = x_ref[...]
    t = t_ref[...]
    col = jax.lax.broadcasted_iota(jnp.int32, (_BR, _BC), 1) + j * _BC
    w = jnp.where(col == t, wc_ref[...], wf_ref[...])
    out_ref[0, 0] += jnp.sum(x * w)

    @pl.when(j == 0)
    def _col0():
        out_ref[0, 0] += jnp.sum(mf_ref[...] * (_C_ROW + _EPS * x[:, 0:1]))


def kernel(x, target):
    t2 = target.reshape(_N_TOK, 1)
    maskf = (t2 != 0).astype(jnp.float32)
    wf = -_EPS * maskf
    wc = -_CONF * maskf
    out = pl.pallas_call(
        _loss_body,
        grid=(_N_TOK // _BR, _SIZE // _BC),
        in_specs=[
            pl.BlockSpec((_BR, _BC), lambda i, j: (i, j)),
            pl.BlockSpec((_BR, 1), lambda i, j: (i, 0)),
            pl.BlockSpec((_BR, 1), lambda i, j: (i, 0)),
            pl.BlockSpec((_BR, 1), lambda i, j: (i, 0)),
            pl.BlockSpec((_BR, 1), lambda i, j: (i, 0)),
        ],
        out_specs=pl.BlockSpec((1, 1), lambda i, j: (0, 0),
                               memory_space=pltpu.SMEM),
        out_shape=jax.ShapeDtypeStruct((1, 1), jnp.float32),
    )(x, t2, wf, wc, maskf)
    return out[0, 0]
